# baseline (speedup 1.0000x reference)
import functools

import jax
import jax.numpy as jnp
from jax import lax
from jax.experimental import pallas as pl
from jax.experimental.pallas import tpu as pltpu
from jax.experimental.pallas import tpu_sc as plsc

F = 16
B = 4096
N = 1000000
NC = 2
NS = 16
NW = NC * NS
BPW = B // NW

TAIL = (N // 128 - 1) * 128
MAIN_LIM = TAIL + 128
TW = N - TAIL


def _i16(x):
    return jnp.full((16,), x, dtype=jnp.int32)


@functools.partial(
    pl.kernel,
    out_type=(
        jax.ShapeDtypeStruct((F, B), jnp.float32),
        jax.ShapeDtypeStruct((F, B), jnp.float32),
    ),
    mesh=plsc.VectorSubcoreMesh(core_axis_name="c", subcore_axis_name="s"),
    compiler_params=pltpu.CompilerParams(needs_layout_passes=False),
    scratch_types=[
        pltpu.VMEM((BPW,), jnp.int32),
        pltpu.VMEM((BPW,), jnp.int32),
        pltpu.VMEM((F, BPW), jnp.float32),
        pltpu.VMEM((F, BPW), jnp.float32),
        pltpu.VMEM((F, TW), jnp.float32),
    ]
    + [pltpu.VMEM((F, 128), jnp.float32) for _ in range(32)]
    + [pltpu.SemaphoreType.DMA],
)
def _sc_factors(users_hbm, items_hbm, uft_hbm, itft_hbm,
                uft_out, itft_out,
                uidx_v, iidx_v, fu_v, fi_v, tail_v, *slabs_and_sem):
    bank = (slabs_and_sem[:16], slabs_and_sem[16:32])
    sem = slabs_and_sem[32]
    iota = lax.iota(jnp.int32, 16)
    wid = lax.axis_index("s") * NC + lax.axis_index("c")
    base = wid * BPW
    pltpu.sync_copy(users_hbm.at[pl.ds(base, BPW)], uidx_v)
    pltpu.sync_copy(items_hbm.at[pl.ds(base, BPW)], iidx_v)

    NG = BPW // 16

    def gather_table(tab_hbm, idx_v, out_v):
        pltpu.sync_copy(tab_hbm.at[:, pl.ds(TAIL, TW)], tail_v)

        def fire(g, slabs):
            vec = idx_v[pl.ds(g * 16, 16)]
            off_vec = jnp.minimum(vec & _i16(~127), _i16(TAIL))
            for k in range(16):
                off = pl.multiple_of(off_vec[k], 128)
                pltpu.async_copy(tab_hbm.at[:, pl.ds(off, 128)], slabs[k], sem)

        def drain_extract(g, slabs):
            j0 = g * 16
            vec = idx_v[pl.ds(j0, 16)]
            off_vec = jnp.minimum(vec & _i16(~127), _i16(TAIL))
            cm_vec = jnp.minimum(vec - off_vec, _i16(127))
            ct_vec = jnp.minimum(
                jnp.maximum(vec - _i16(TAIL), _i16(0)), _i16(TW - 1))
            for k in range(16):
                pltpu.make_async_copy(tab_hbm.at[:, pl.ds(0, 128)], slabs[k],
                                      sem).wait()
            for k in range(16):
                col_main = plsc.load_gather(slabs[k], [iota, _i16(cm_vec[k])])
                col_tail = plsc.load_gather(tail_v, [iota, _i16(ct_vec[k])])
                pred = _i16(vec[k]) < _i16(MAIN_LIM)
                col = jnp.where(pred, col_main, col_tail)
                plsc.store_scatter(out_v, [iota, _i16(j0 + k)], col)

        fire(0, bank[0])

        @pl.loop(0, NG // 2 - 1, unroll=1)
        def _pipe(p):
            g = p * 2
            fire(g + 1, bank[1])
            drain_extract(g, bank[0])
            fire(g + 2, bank[0])
            drain_extract(g + 1, bank[1])

        fire(NG - 1, bank[1])
        drain_extract(NG - 2, bank[0])
        drain_extract(NG - 1, bank[1])

    gather_table(uft_hbm, uidx_v, fu_v)
    gather_table(itft_hbm, iidx_v, fi_v)

    pltpu.sync_copy(fu_v, uft_out.at[:, pl.ds(base, BPW)])
    pltpu.sync_copy(fi_v, itft_out.at[:, pl.ds(base, BPW)])


@functools.partial(
    pl.kernel,
    out_type=(
        jax.ShapeDtypeStruct((B,), jnp.float32),
        jax.ShapeDtypeStruct((B,), jnp.float32),
    ),
    mesh=plsc.VectorSubcoreMesh(core_axis_name="c", subcore_axis_name="s"),
    compiler_params=pltpu.CompilerParams(needs_layout_passes=False),
    scratch_types=[
        pltpu.VMEM((BPW,), jnp.int32),
        pltpu.VMEM((BPW,), jnp.int32),
        pltpu.VMEM((BPW,), jnp.float32),
        pltpu.VMEM((BPW,), jnp.float32),
        pltpu.VMEM((8 * BPW,), jnp.float32),
        pltpu.SemaphoreType.DMA,
    ],
)
def _sc_bias(users_hbm, items_hbm, ub_hbm, ib_hbm,
             ub_out, ib_out,
             uidx_v, iidx_v, bu_v, bi_v, braw_v, sem):
    iota = lax.iota(jnp.int32, 16)
    wid = lax.axis_index("s") * NC + lax.axis_index("c")
    base = wid * BPW
    pltpu.sync_copy(users_hbm.at[pl.ds(base, BPW)], uidx_v)
    pltpu.sync_copy(items_hbm.at[pl.ds(base, BPW)], iidx_v)

    NG = BPW // 16

    def gather_bias(bias_hbm, idx_v, bout_v):
        @pl.loop(0, NG, unroll=1)
        def _fire(g):
            j0 = g * 16
            vec = idx_v[pl.ds(j0, 16)]
            boff_vec = vec & _i16(~7)
            for k in range(16):
                boff = pl.multiple_of(boff_vec[k], 8)
                pltpu.async_copy(bias_hbm.at[pl.ds(boff, 8)],
                                 braw_v.at[pl.ds((j0 + k) * 8, 8)], sem)

        pltpu.make_async_copy(bias_hbm.at[pl.ds(0, 8 * BPW)], braw_v,
                              sem).wait()

        @pl.loop(0, NG, unroll=1)
        def _extract(g):
            j0 = g * 16
            vec = idx_v[pl.ds(j0, 16)]
            flat = (_i16(j0) + iota) * _i16(8) + (vec & _i16(7))
            bout_v[pl.ds(j0, 16)] = plsc.load_gather(braw_v, [flat])

    gather_bias(ub_hbm, uidx_v, bu_v)
    gather_bias(ib_hbm, iidx_v, bi_v)
    pltpu.sync_copy(bu_v, ub_out.at[pl.ds(base, BPW)])
    pltpu.sync_copy(bi_v, ib_out.at[pl.ds(base, BPW)])


BM = 512


def _mm_body(uft_ref, itft_ref, ub_ref, ib_ref, o_ref):
    acc = lax.dot_general(
        uft_ref[...], itft_ref[...],
        dimension_numbers=(((0,), (0,)), ((), ())),
        preferred_element_type=jnp.float32,
    )
    o_ref[...] = acc + ub_ref[...] + ib_ref[...]


def _tc_matmul(uft, itft, ub_col, ib_row):
    return pl.pallas_call(
        _mm_body,
        grid=(B // BM,),
        in_specs=[
            pl.BlockSpec((F, BM), lambda i: (0, i)),
            pl.BlockSpec((F, B), lambda i: (0, 0)),
            pl.BlockSpec((BM, 1), lambda i: (i, 0)),
            pl.BlockSpec((1, B), lambda i: (0, 0)),
        ],
        out_specs=pl.BlockSpec((BM, B), lambda i: (i, 0)),
        out_shape=jax.ShapeDtypeStruct((B, B), jnp.float32),
    )(uft, itft, ub_col, ib_row)


def kernel(users, items, user_factors, item_factors, user_biases, item_biases):
    users = users.astype(jnp.int32)
    items = items.astype(jnp.int32)
    ub, ib = _sc_bias(users, items,
                      user_biases.reshape(-1), item_biases.reshape(-1))
    uft, itft = _sc_factors(users, items, user_factors.T, item_factors.T)
    return _tc_matmul(uft, itft, ub.reshape(B, 1), ib.reshape(1, B))


# speedup vs baseline: 1.0360x; 1.0360x over previous
import functools

import jax
import jax.numpy as jnp
from jax import lax
from jax.experimental import pallas as pl
from jax.experimental.pallas import tpu as pltpu
from jax.experimental.pallas import tpu_sc as plsc

F = 16
B = 4096
N = 1000000
NC = 2
NS = 16
NW = NC * NS
BPW = B // NW

TAIL = (N // 128 - 1) * 128
MAIN_LIM = TAIL + 128
TW = N - TAIL


def _i16(x):
    return jnp.full((16,), x, dtype=jnp.int32)


@functools.partial(
    pl.kernel,
    out_type=(
        jax.ShapeDtypeStruct((F, B), jnp.float32),
        jax.ShapeDtypeStruct((F, B), jnp.float32),
        jax.ShapeDtypeStruct((B,), jnp.float32),
        jax.ShapeDtypeStruct((B,), jnp.float32),
    ),
    mesh=plsc.VectorSubcoreMesh(core_axis_name="c", subcore_axis_name="s"),
    compiler_params=pltpu.CompilerParams(needs_layout_passes=False),
    scratch_types=[
        pltpu.VMEM((BPW,), jnp.int32),
        pltpu.VMEM((BPW,), jnp.int32),
        pltpu.VMEM((F, BPW), jnp.float32),
        pltpu.VMEM((F, BPW), jnp.float32),
        pltpu.VMEM((BPW,), jnp.float32),
        pltpu.VMEM((BPW,), jnp.float32),
        pltpu.VMEM((8 * BPW,), jnp.float32),
        pltpu.VMEM((F, TW), jnp.float32),
    ]
    + [pltpu.VMEM((F, 128), jnp.float32) for _ in range(32)]
    + [pltpu.SemaphoreType.DMA, pltpu.SemaphoreType.DMA],
)
def _sc_gather(users_hbm, items_hbm, uft_hbm, itft_hbm, ub_hbm, ib_hbm,
               uft_out, itft_out, ub_out, ib_out,
               uidx_v, iidx_v, fu_v, fi_v, bu_v, bi_v, braw_v, tail_v,
               *slabs_and_sem):
    bank = (slabs_and_sem[:16], slabs_and_sem[16:32])
    sem, bsem = slabs_and_sem[32], slabs_and_sem[33]
    iota = lax.iota(jnp.int32, 16)
    wid = lax.axis_index("s") * NC + lax.axis_index("c")
    base = wid * BPW
    pltpu.sync_copy(users_hbm.at[pl.ds(base, BPW)], uidx_v)
    pltpu.sync_copy(items_hbm.at[pl.ds(base, BPW)], iidx_v)

    NG = BPW // 16

    def gather_table(tab_hbm, bias_hbm, idx_v, out_v, bout_v):
        pltpu.sync_copy(tab_hbm.at[:, pl.ds(TAIL, TW)], tail_v)

        @pl.loop(0, NG, unroll=1)
        def _bias_fire(g):
            j0 = g * 16
            vec = idx_v[pl.ds(j0, 16)]
            boff_vec = vec & _i16(~7)
            for k in range(16):
                boff = pl.multiple_of(boff_vec[k], 8)
                pltpu.async_copy(bias_hbm.at[pl.ds(boff, 8)],
                                 braw_v.at[pl.ds((j0 + k) * 8, 8)], bsem)

        def fire(g, slabs):
            vec = idx_v[pl.ds(g * 16, 16)]
            off_vec = jnp.minimum(vec & _i16(~127), _i16(TAIL))
            for k in range(16):
                off = pl.multiple_of(off_vec[k], 128)
                pltpu.async_copy(tab_hbm.at[:, pl.ds(off, 128)], slabs[k], sem)

        def drain_extract(g, slabs):
            j0 = g * 16
            vec = idx_v[pl.ds(j0, 16)]
            off_vec = jnp.minimum(vec & _i16(~127), _i16(TAIL))
            cm_vec = jnp.minimum(vec - off_vec, _i16(127))
            ct_vec = jnp.minimum(
                jnp.maximum(vec - _i16(TAIL), _i16(0)), _i16(TW - 1))
            for k in range(16):
                pltpu.make_async_copy(tab_hbm.at[:, pl.ds(0, 128)], slabs[k],
                                      sem).wait()
            for k in range(16):
                col_main = plsc.load_gather(slabs[k], [iota, _i16(cm_vec[k])])
                col_tail = plsc.load_gather(tail_v, [iota, _i16(ct_vec[k])])
                pred = _i16(vec[k]) < _i16(MAIN_LIM)
                col = jnp.where(pred, col_main, col_tail)
                plsc.store_scatter(out_v, [iota, _i16(j0 + k)], col)

        fire(0, bank[0])

        @pl.loop(0, NG // 2 - 1, unroll=1)
        def _pipe(p):
            g = p * 2
            fire(g + 1, bank[1])
            drain_extract(g, bank[0])
            fire(g + 2, bank[0])
            drain_extract(g + 1, bank[1])

        fire(NG - 1, bank[1])
        drain_extract(NG - 2, bank[0])
        drain_extract(NG - 1, bank[1])

        pltpu.make_async_copy(bias_hbm.at[pl.ds(0, 8 * BPW)], braw_v,
                              bsem).wait()

        @pl.loop(0, NG, unroll=1)
        def _bias_extract(g):
            j0 = g * 16
            vec = idx_v[pl.ds(j0, 16)]
            flat = (_i16(j0) + iota) * _i16(8) + (vec & _i16(7))
            bout_v[pl.ds(j0, 16)] = plsc.load_gather(braw_v, [flat])

    gather_table(uft_hbm, ub_hbm, uidx_v, fu_v, bu_v)
    gather_table(itft_hbm, ib_hbm, iidx_v, fi_v, bi_v)

    pltpu.sync_copy(fu_v, uft_out.at[:, pl.ds(base, BPW)])
    pltpu.sync_copy(fi_v, itft_out.at[:, pl.ds(base, BPW)])
    pltpu.sync_copy(bu_v, ub_out.at[pl.ds(base, BPW)])
    pltpu.sync_copy(bi_v, ib_out.at[pl.ds(base, BPW)])


BM = 512


def _mm_body(uft_ref, itft_ref, ub_ref, ib_ref, o_ref):
    acc = lax.dot_general(
        uft_ref[...], itft_ref[...],
        dimension_numbers=(((0,), (0,)), ((), ())),
        preferred_element_type=jnp.float32,
    )
    o_ref[...] = acc + ub_ref[...] + ib_ref[...]


def _tc_matmul(uft, itft, ub_col, ib_row):
    return pl.pallas_call(
        _mm_body,
        grid=(B // BM,),
        in_specs=[
            pl.BlockSpec((F, BM), lambda i: (0, i)),
            pl.BlockSpec((F, B), lambda i: (0, 0)),
            pl.BlockSpec((BM, 1), lambda i: (i, 0)),
            pl.BlockSpec((1, B), lambda i: (0, 0)),
        ],
        out_specs=pl.BlockSpec((BM, B), lambda i: (i, 0)),
        out_shape=jax.ShapeDtypeStruct((B, B), jnp.float32),
    )(uft, itft, ub_col, ib_row)


def kernel(users, items, user_factors, item_factors, user_biases, item_biases):
    uft, itft, ub, ib = _sc_gather(
        users.astype(jnp.int32), items.astype(jnp.int32),
        user_factors.T, item_factors.T,
        user_biases.reshape(-1), item_biases.reshape(-1))
    return _tc_matmul(uft, itft, ub.reshape(B, 1), ib.reshape(1, B))
